# baseline (speedup 1.0000x reference)
import functools

import jax
import jax.numpy as jnp
from jax import lax
from jax.experimental import pallas as pl
from jax.experimental.pallas import tpu as pltpu
from jax.experimental.pallas import tpu_sc as plsc

EMBED_DIM = 64
CHUNK = 128
NBUF = 4


@functools.lru_cache(maxsize=None)
def _build(n_rows):
    info = plsc.get_sparse_core_info()
    nc, ns = info.num_cores, info.num_subcores
    nw = nc * ns
    per_w = n_rows // nw
    assert per_w * nw == n_rows and per_w % CHUNK == 0
    n_chunks = per_w // CHUNK
    assert n_chunks % NBUF == 0
    rounds = n_chunks // NBUF
    mesh = plsc.VectorSubcoreMesh(core_axis_name="c", subcore_axis_name="s")

    @functools.partial(
        pl.kernel,
        out_type=jax.ShapeDtypeStruct((n_rows, EMBED_DIM), jnp.float32),
        mesh=mesh,
        compiler_params=pltpu.CompilerParams(use_tc_tiling_on_sc=False),
        scratch_types=[
            pltpu.VMEM((n_chunks, CHUNK), jnp.int32),
            *[pltpu.VMEM((CHUNK, EMBED_DIM), jnp.float32) for _ in range(NBUF)],
            *[pltpu.SemaphoreType.DMA for _ in range(2 * NBUF)],
        ],
    )
    def k(idx_hbm, table_hbm, out_hbm, idx_v, *rest):
        bufs = rest[:NBUF]
        gsems = rest[NBUF:2 * NBUF]
        ssems = rest[2 * NBUF:]
        wid = lax.axis_index("s") * nc + lax.axis_index("c")
        crow0 = wid * n_chunks

        pltpu.sync_copy(idx_hbm.at[pl.ds(crow0, n_chunks)], idx_v)

        def gather(g, b):
            return pltpu.make_async_copy(
                table_hbm.at[idx_v.at[g]], bufs[b], gsems[b])

        def scatter(g, b):
            row0 = (crow0 + g) * CHUNK
            return pltpu.make_async_copy(
                bufs[b], out_hbm.at[pl.ds(row0, CHUNK)], ssems[b])

        for b in range(NBUF):
            gather(b, b).start()

        def round_body(r, carry):
            for b in range(NBUF):
                g = r * NBUF + b
                gather(g, b).wait()
                scatter(g, b).start()
            for b in range(NBUF):
                g = r * NBUF + b
                scatter(g, b).wait()
                gather(g + NBUF, b).start()
            return carry

        lax.fori_loop(0, rounds - 1, round_body, 0)

        last = (rounds - 1) * NBUF
        for b in range(NBUF):
            gather(last + b, b).wait()
            scatter(last + b, b).start()
        for b in range(NBUF):
            scatter(last + b, b).wait()

    return k


def kernel(x, table):
    batch, hist = x.shape
    n_rows = batch * hist
    idx = x.reshape(n_rows // CHUNK, CHUNK).astype(jnp.int32)
    out = _build(n_rows)(idx, table)
    return out.reshape(batch, hist, EMBED_DIM)


# speedup vs baseline: 1.1821x; 1.1821x over previous
import functools

import jax
import jax.numpy as jnp
import numpy as np
from jax import lax
from jax.experimental import pallas as pl
from jax.experimental.pallas import tpu as pltpu
from jax.experimental.pallas import tpu_sc as plsc

EMBED_DIM = 64
CHUNK = 128
NBUF = 4
PAD_H = 24


@functools.lru_cache(maxsize=None)
def _build(n_rows, hist):
    info = plsc.get_sparse_core_info()
    nc, ns = info.num_cores, info.num_subcores
    nw = nc * ns
    per_w = n_rows // nw
    assert per_w * nw == n_rows and per_w % CHUNK == 0
    n_chunks = per_w // CHUNK
    assert n_chunks % NBUF == 0
    rounds = n_chunks // NBUF
    n_batches = n_rows // hist
    out_rows = n_batches * PAD_H * 2
    mesh = plsc.VectorSubcoreMesh(core_axis_name="c", subcore_axis_name="s")

    @functools.partial(
        pl.kernel,
        out_type=jax.ShapeDtypeStruct((out_rows, EMBED_DIM), jnp.float32),
        mesh=mesh,
        compiler_params=pltpu.CompilerParams(use_tc_tiling_on_sc=False),
        scratch_types=[
            pltpu.VMEM((n_chunks, CHUNK), jnp.int32),
            pltpu.VMEM((n_chunks, CHUNK), jnp.int32),
            *[pltpu.VMEM((CHUNK, EMBED_DIM), jnp.float32) for _ in range(NBUF)],
            *[pltpu.SemaphoreType.DMA for _ in range(2 * NBUF)],
        ],
    )
    def k(idx_hbm, didx_hbm, table_hbm, out_hbm, idx_v, didx_v, *rest):
        bufs = rest[:NBUF]
        gsems = rest[NBUF:2 * NBUF]
        ssems = rest[2 * NBUF:]
        wid = lax.axis_index("s") * nc + lax.axis_index("c")
        crow0 = wid * n_chunks

        pltpu.sync_copy(idx_hbm.at[pl.ds(crow0, n_chunks)], idx_v)
        pltpu.sync_copy(didx_hbm.at[pl.ds(crow0, n_chunks)], didx_v)

        def gather(g, b):
            return pltpu.make_async_copy(
                table_hbm.at[idx_v.at[g]], bufs[b], gsems[b])

        def scatter(g, b):
            return pltpu.make_async_copy(
                bufs[b], out_hbm.at[didx_v.at[g]], ssems[b])

        for b in range(NBUF):
            gather(b, b).start()

        def round_body(r, carry):
            for b in range(NBUF):
                g = r * NBUF + b
                gather(g, b).wait()
                scatter(g, b).start()
            for b in range(NBUF):
                g = r * NBUF + b
                scatter(g, b).wait()
                gather(g + NBUF, b).start()
            return carry

        lax.fori_loop(0, rounds - 1, round_body, 0)

        last = (rounds - 1) * NBUF
        for b in range(NBUF):
            gather(last + b, b).wait()
            scatter(last + b, b).start()
        for b in range(NBUF):
            scatter(last + b, b).wait()

    return k


def kernel(x, table):
    batch, hist = x.shape
    n_rows = batch * hist
    idx = x.reshape(n_rows // CHUNK, CHUNK)
    n = np.arange(n_rows, dtype=np.int64)
    didx_np = 2 * (PAD_H * (n // hist) + n % hist)
    didx = jnp.asarray(
        didx_np.reshape(n_rows // CHUNK, CHUNK).astype(np.int32))
    out = _build(n_rows, hist)(idx, didx, table)
    out3 = out.reshape(batch, PAD_H, 2 * EMBED_DIM)
    return out3[:, :hist, :EMBED_DIM]
